# baseline (speedup 1.0000x reference)
import functools

import jax
import jax.numpy as jnp
from jax import lax
from jax.experimental import pallas as pl
from jax.experimental.pallas import tpu as pltpu
from jax.experimental.pallas import tpu_sc as plsc

_V = 1000
_D = 64
_B = 1024
_T = 50
_C = 16


def _table_body(embed_ref, head_ref, out_ref):
    out_ref[...] = lax.dot_general(
        embed_ref[...], head_ref[...],
        dimension_numbers=(((1,), (1,)), ((), ())),
        preferred_element_type=jnp.float32,
    )


def _make_logits_table(embed_table, head_w):
    return pl.pallas_call(
        _table_body,
        out_shape=jax.ShapeDtypeStruct((_V, _V), jnp.float32),
    )(embed_table, head_w)


def _gather_transpose(table, idx_t):
    info = plsc.get_sparse_core_info()
    nc, ns = info.num_cores, info.num_subcores
    nw = nc * ns
    n_chunks = _T * (_B // _C)
    cpw = n_chunks // nw
    hpt = _B // _C
    hpb = 128 // _C

    mesh = plsc.VectorSubcoreMesh(core_axis_name="c", subcore_axis_name="s")

    @functools.partial(
        pl.kernel,
        out_type=jax.ShapeDtypeStruct((_T, _V // 8, 8, 8, 128), jnp.float32),
        mesh=mesh,
        compiler_params=pltpu.CompilerParams(use_tc_tiling_on_sc=False,
                                             needs_layout_passes=False),
        scratch_types=[
            pltpu.VMEM((2, _C), jnp.int32),
            pltpu.VMEM((2, _C, _V), jnp.float32),
            pltpu.VMEM((2, _V // 8, 8, _C), jnp.float32),
            pltpu.SemaphoreType.DMA,
            pltpu.SemaphoreType.DMA,
        ],
    )
    def k(table_hbm, idx_hbm, out_hbm, idx2, gbuf2, tbuf2, gsem, wsem):
        c = lax.axis_index("c")
        s = lax.axis_index("s")
        wid = s * nc + c
        c0 = wid * cpw

        lane16 = lax.iota(jnp.int32, _C)

        def coords(i):
            g = c0 + i
            t = lax.div(g, hpt)
            r = lax.rem(g, hpt)
            tb = lax.div(r, hpb)
            h = lax.rem(r, hpb)
            return t, tb, h

        def gather(i, p):
            t, tb, h = coords(i)
            pltpu.sync_copy(idx_hbm.at[t, pl.ds(128 * tb + _C * h, _C)],
                            idx2.at[p])
            pltpu.async_copy(table_hbm.at[idx2.at[p]], gbuf2.at[p], gsem)

        def wait_gather(p):
            pltpu.make_async_copy(table_hbm.at[pl.ds(0, _C)], gbuf2.at[p],
                                  gsem).wait()

        def transpose(p):
            pv = jnp.full((_C,), p, dtype=jnp.int32)

            def body(tv, carry):
                for sub in range(8):
                    v = jnp.full((_C,), 8 * tv + sub, dtype=jnp.int32)
                    vec = plsc.load_gather(gbuf2.at[p], [lane16, v])
                    tbuf2[p, tv, sub, :] = vec
                return carry

            lax.fori_loop(0, _V // 8, body, 0)

        def write(i, p):
            t, tb, h = coords(i)
            pltpu.async_copy(tbuf2.at[p],
                             out_hbm.at[t, :, tb, :, pl.ds(_C * h, _C)], wsem)

        def wait_write(i, p):
            t, tb, h = coords(i)
            pltpu.make_async_copy(tbuf2.at[p],
                                  out_hbm.at[t, :, tb, :, pl.ds(_C * h, _C)],
                                  wsem).wait()

        gather(0, 0)
        gather(1, 1)
        wait_gather(0)
        transpose(0)
        write(0, 0)

        def body(i, carry):
            p = lax.rem(i, 2)
            q = 1 - p

            @pl.when(i <= cpw - 1)
            def _():
                gather(i, p)

            wait_gather(q)

            @pl.when(i >= 3)
            def _():
                wait_write(i - 3, q)

            transpose(q)
            write(i - 1, q)
            return carry

        lax.fori_loop(2, cpw + 1, body, 0)

        wait_write(cpw - 2, lax.rem(cpw - 2, 2))
        wait_write(cpw - 1, lax.rem(cpw - 1, 2))

    return k(table, idx_t)


def kernel(input_ids, embed_table, head_w):
    table = _make_logits_table(embed_table, head_w)
    idx_t = input_ids.astype(jnp.int32).T
    out5 = _gather_transpose(table, idx_t)
    y = out5.transpose(0, 1, 3, 2, 4).reshape(_T, _V, _B)
    return y.transpose(2, 0, 1)


# speedup vs baseline: 2.9017x; 2.9017x over previous
import functools

import jax
import jax.numpy as jnp
from jax import lax
from jax.experimental import pallas as pl
from jax.experimental.pallas import tpu as pltpu
from jax.experimental.pallas import tpu_sc as plsc

_V = 1000
_D = 64
_B = 1024
_T = 50
_C = 16


def _table_body(embed_ref, head_ref, out_ref):
    out_ref[...] = lax.dot_general(
        embed_ref[...], head_ref[...],
        dimension_numbers=(((1,), (1,)), ((), ())),
        preferred_element_type=jnp.float32,
    )


def _make_logits_table(embed_table, head_w):
    return pl.pallas_call(
        _table_body,
        out_shape=jax.ShapeDtypeStruct((_V, _V), jnp.float32),
    )(embed_table, head_w)


def _gather_transpose(table, idx_t):
    info = plsc.get_sparse_core_info()
    nc, ns = info.num_cores, info.num_subcores
    nw = nc * ns
    n_chunks = _T * (_B // _C)
    cpw = n_chunks // nw
    hpt = _B // _C
    hpb = 128 // _C

    mesh = plsc.VectorSubcoreMesh(core_axis_name="c", subcore_axis_name="s")

    @functools.partial(
        pl.kernel,
        out_type=jax.ShapeDtypeStruct((_T, _V // 8, 8, 8, 128), jnp.float32),
        mesh=mesh,
        compiler_params=pltpu.CompilerParams(use_tc_tiling_on_sc=False,
                                             needs_layout_passes=False),
        scratch_types=[
            pltpu.VMEM((2, _C), jnp.int32),
            pltpu.VMEM((2, _C, _V), jnp.float32),
            pltpu.VMEM((2, _V // 8, 8, _C), jnp.float32),
            pltpu.SemaphoreType.DMA,
            pltpu.SemaphoreType.DMA,
        ],
    )
    def k(table_hbm, idx_hbm, out_hbm, idx2, gbuf2, tbuf2, gsem, wsem):
        c = lax.axis_index("c")
        s = lax.axis_index("s")
        wid = s * nc + c
        c0 = wid * cpw

        lane16 = lax.iota(jnp.int32, _C)

        def coords(i):
            g = c0 + i
            t = lax.div(g, hpt)
            r = lax.rem(g, hpt)
            tb = lax.div(r, hpb)
            h = lax.rem(r, hpb)
            return t, tb, h

        def gather(i, p):
            t, tb, h = coords(i)
            pltpu.sync_copy(idx_hbm.at[t, pl.ds(128 * tb + _C * h, _C)],
                            idx2.at[p])
            pltpu.async_copy(table_hbm.at[idx2.at[p]], gbuf2.at[p], gsem)

        def wait_gather(p):
            pltpu.make_async_copy(table_hbm.at[pl.ds(0, _C)], gbuf2.at[p],
                                  gsem).wait()

        def transpose(p):
            @functools.partial(plsc.parallel_loop, 0, _V // 8, unroll=4)
            def _body(tv):
                for sub in range(8):
                    v = jnp.full((_C,), 8 * tv + sub, dtype=jnp.int32)
                    vec = plsc.load_gather(gbuf2.at[p], [lane16, v])
                    tbuf2[p, tv, sub, :] = vec

        def write(i, p):
            t, tb, h = coords(i)
            pltpu.async_copy(tbuf2.at[p],
                             out_hbm.at[t, :, tb, :, pl.ds(_C * h, _C)], wsem)

        def wait_write(i, p):
            t, tb, h = coords(i)
            pltpu.make_async_copy(tbuf2.at[p],
                                  out_hbm.at[t, :, tb, :, pl.ds(_C * h, _C)],
                                  wsem).wait()

        gather(0, 0)
        gather(1, 1)
        wait_gather(0)
        transpose(0)
        write(0, 0)

        def body(i, carry):
            p = lax.rem(i, 2)
            q = 1 - p

            @pl.when(i <= cpw - 1)
            def _():
                gather(i, p)

            wait_gather(q)

            @pl.when(i >= 3)
            def _():
                wait_write(i - 3, q)

            transpose(q)
            write(i - 1, q)
            return carry

        lax.fori_loop(2, cpw + 1, body, 0)

        wait_write(cpw - 2, lax.rem(cpw - 2, 2))
        wait_write(cpw - 1, lax.rem(cpw - 1, 2))

    return k(table, idx_t)


def kernel(input_ids, embed_table, head_w):
    table = _make_logits_table(embed_table, head_w)
    idx_t = input_ids.astype(jnp.int32).T
    out5 = _gather_transpose(table, idx_t)
    y = out5.transpose(0, 1, 3, 2, 4).reshape(_T, _V, _B)
    return y.transpose(2, 0, 1)
